# baseline (speedup 1.0000x reference)
import functools

import jax
import jax.numpy as jnp
import numpy as np
from jax.experimental import pallas as pl
import jax.experimental.pallas.tpu as pltpu
from jax.experimental.pallas import tpu_sc as plsc

ALPHA = 1.0
BETA = 1.0
N_TDA = 16
Q = 2
NN_SIZE = 5
RADIUS = 0.07
H = 0.03
EPS = 1e-12
BIG = 1e9

_T = 256


def _loss_kernel(ptile_ref, pfull_ref, gfull_ref,
                 pcx_ref, pcy_ref, pcz_ref,
                 prx_ref, pry_ref, prz_ref,
                 grx_ref, gry_ref, grz_ref,
                 pbb_ref, gbb_ref, pcd_ref,
                 emd_ref, rep_ref):
    b = pl.program_id(0)
    i = pl.program_id(1)
    B = pl.num_programs(0)
    NT = pl.num_programs(1)
    N = NT * _T

    @pl.when((b == 0) & (i == 0))
    def _init():
        zero = jnp.zeros((1, 1), jnp.float32)
        emd_ref[...] = zero
        rep_ref[...] = zero

    pt = ptile_ref[0]
    pc = [pcx_ref[0], pcy_ref[0], pcz_ref[0]]
    pr = [prx_ref[0], pry_ref[0], prz_ref[0]]
    gr = [grx_ref[0], gry_ref[0], grz_ref[0]]

    aa = jnp.sum(pt * pt, axis=1, keepdims=True)
    ciota = jax.lax.broadcasted_iota(jnp.int32, (_T, N), 1)

    def sel_matrix(full_ref, bb_ref):
        ab = jax.lax.dot_general(pt, full_ref[0], (((1,), (1,)), ((), ())),
                                 preferred_element_type=jnp.float32)
        return (aa - 2.0 * ab) + bb_ref[0]

    def exact_d2(rows):
        acc = None
        for c in range(3):
            t = (pc[c] - rows[c]) ** 2
            acc = t if acc is None else acc + t
        return acc

    def first_min_idx(d, m):
        return jnp.min(jnp.where(d == m, ciota, N), axis=1, keepdims=True)

    def value_at(d_exact, idx):
        return jnp.sum(jnp.where(ciota == idx, d_exact, 0.0),
                       axis=1, keepdims=True)

    d_pg_sel = sel_matrix(gfull_ref, gbb_ref)
    d_pg_exact = exact_d2(gr)
    m = jnp.min(d_pg_sel, axis=1, keepdims=True)
    idx = first_min_idx(d_pg_sel, m)
    val = value_at(d_pg_exact, idx)
    mnsum = jnp.sum(val, axis=0, keepdims=True)
    emd_ref[...] += mnsum * (100.0 / (3.0 * N * B)) / pcd_ref[0]

    d_pp_sel = sel_matrix(pfull_ref, pbb_ref)
    d_pp_exact = exact_d2(pr)
    rep_acc = jnp.zeros((1, 1), jnp.float32)
    for k in range(NN_SIZE):
        m = jnp.min(d_pp_sel, axis=1, keepdims=True)
        idxm = first_min_idx(d_pp_sel, m)
        if k > 0:
            val = value_at(d_pp_exact, idxm)
            d2c = jnp.maximum(val, EPS)
            dist = jnp.sqrt(d2c)
            w = jnp.exp(-d2c / (H * H))
            rep_acc = rep_acc + jnp.sum((RADIUS - dist) * w, axis=0,
                                        keepdims=True)
        if k < NN_SIZE - 1:
            d_pp_sel = jnp.where(ciota == idxm, BIG, d_pp_sel)
    rep_ref[...] += rep_acc / (B * N * (NN_SIZE - 1))


def _oddeven_pairs(n):
    pairs = []
    p = 1
    while p < n:
        k = p
        while k >= 1:
            for j in range(k % p, n - k, 2 * k):
                for i in range(0, min(k, n - j - k)):
                    if (i + j) // (2 * p) == (i + j + k) // (2 * p):
                        pairs.append((i + j, i + j + k))
            k //= 2
        p *= 2
    return pairs


_PAIRS16 = _oddeven_pairs(16)


def _nsqrt(x):
    b = plsc.bitcast(x, jnp.int32)
    y = plsc.bitcast((b >> 1) + 0x1fbd1df5, jnp.float32)
    for _ in range(4):
        y = 0.5 * (y + x / y)
    return y


def _sorted_diag(buf):
    px = buf[0, :]
    py = buf[1, :]
    pz = buf[2, :]
    iot = jax.lax.iota(jnp.int32, 16)
    rows = []
    for i in range(N_TDA - 1):
        idx = jnp.full((16,), i, jnp.int32)
        pxi = px.at[idx].get(mode="promise_in_bounds")
        pyi = py.at[idx].get(mode="promise_in_bounds")
        pzi = pz.at[idx].get(mode="promise_in_bounds")
        dx = px - pxi
        dy = py - pyi
        dz = pz - pzi
        d2 = dx * dx + dy * dy + dz * dz
        dist = _nsqrt(jnp.maximum(d2, EPS))
        rows.append(jnp.where(iot > i, dist, BIG))
    rows.append(jnp.full((16,), BIG, jnp.float32))
    rows = [jnp.sort(r) for r in rows]
    for (a, b) in _PAIRS16:
        u, v = rows[a], rows[b]
        vr = jax.lax.rev(v, (0,))
        rows[a] = jnp.sort(jnp.minimum(u, vr))
        rows[b] = jnp.sort(jnp.maximum(u, vr))
    return rows


def _tda_body(p_hbm, g_hbm, out_hbm):
    c = jax.lax.axis_index("c")
    s = jax.lax.axis_index("s")

    def scoped(pbuf, gbuf, obuf):
        @pl.when((c == 0) & (s < 8))
        def _():
            pltpu.sync_copy(p_hbm.at[s], pbuf)
            pltpu.sync_copy(g_hbm.at[s], gbuf)
            dp = _sorted_diag(pbuf)
            dg = _sorted_diag(gbuf)
            acc = jnp.zeros((16,), jnp.float32)
            for k in range(16):
                d = dp[k] - dg[k]
                acc = acc + d * d
            sumsq = jnp.sum(acc)
            tda = _nsqrt(jnp.broadcast_to(sumsq + EPS, (16,)))
            obuf[...] = tda
            pltpu.sync_copy(obuf, out_hbm.at[s])

    pl.run_scoped(scoped,
                  pltpu.VMEM((3, 16), jnp.float32),
                  pltpu.VMEM((3, 16), jnp.float32),
                  pltpu.VMEM((16,), jnp.float32))


@jax.jit
def _run(pred, gt, pcd_radius):
    B, N, _ = pred.shape
    pcols = [pred[:, :, c:c + 1] for c in range(3)]
    prows = [jnp.swapaxes(p, 1, 2) for p in pcols]
    grows = [jnp.swapaxes(gt[:, :, c:c + 1], 1, 2) for c in range(3)]
    pbb = jnp.swapaxes(jnp.sum(pred * pred, axis=-1, keepdims=True), 1, 2)
    gbb = jnp.swapaxes(jnp.sum(gt * gt, axis=-1, keepdims=True), 1, 2)
    grid = (B, N // _T)
    tile_spec = pl.BlockSpec((1, _T, 3), lambda b, i: (b, i, 0))
    full_spec = pl.BlockSpec((1, N, 3), lambda b, i: (b, 0, 0))
    col_spec = pl.BlockSpec((1, _T, 1), lambda b, i: (b, i, 0))
    row_spec = pl.BlockSpec((1, 1, N), lambda b, i: (b, 0, 0))
    scalar_spec = pl.BlockSpec((1, 1), lambda b, i: (0, 0))
    emd, rep = pl.pallas_call(
        _loss_kernel,
        grid=grid,
        in_specs=[tile_spec, full_spec, full_spec] +
                 [col_spec] * 3 + [row_spec] * 6 + [row_spec] * 2 +
                 [pl.BlockSpec((1, 1, 1), lambda b, i: (b, 0, 0))],
        out_specs=[scalar_spec] * 2,
        out_shape=[jax.ShapeDtypeStruct((1, 1), jnp.float32)] * 2,
    )(pred, pred, gt, *pcols, *prows, *grows, pbb, gbb,
      pcd_radius.reshape(B, 1, 1))

    p16t = jnp.swapaxes(pred[:, :N_TDA, :], 1, 2)
    g16t = jnp.swapaxes(gt[:, :N_TDA, :], 1, 2)
    tda_rows = pl.kernel(
        _tda_body,
        out_type=jax.ShapeDtypeStruct((8, 16), jnp.float32),
        mesh=plsc.VectorSubcoreMesh(core_axis_name="c",
                                    subcore_axis_name="s"),
        compiler_params=pltpu.CompilerParams(needs_layout_passes=False),
    )(p16t, g16t)
    tda = jnp.mean(tda_rows[:, 0])

    return (emd[0, 0], rep[0, 0] * ALPHA, tda * BETA)


def kernel(pred, gt, pcd_radius):
    return _run(pred, gt, pcd_radius)


# speedup vs baseline: 1.0373x; 1.0373x over previous
import functools

import jax
import jax.numpy as jnp
import numpy as np
from jax.experimental import pallas as pl
import jax.experimental.pallas.tpu as pltpu
from jax.experimental.pallas import tpu_sc as plsc

ALPHA = 1.0
BETA = 1.0
N_TDA = 16
Q = 2
NN_SIZE = 5
RADIUS = 0.07
H = 0.03
EPS = 1e-12
BIG = 1e9

_T = 256


def _loss_kernel(ptile_ref, pfull_ref, gfull_ref,
                 pcx_ref, pcy_ref, pcz_ref,
                 prx_ref, pry_ref, prz_ref,
                 grx_ref, gry_ref, grz_ref,
                 pbb_ref, gbb_ref, pcd_ref,
                 emd_ref, rep_ref):
    b = pl.program_id(0)
    i = pl.program_id(1)
    B = pl.num_programs(0)
    NT = pl.num_programs(1)
    N = NT * _T

    @pl.when((b == 0) & (i == 0))
    def _init():
        zero = jnp.zeros((1, 1), jnp.float32)
        emd_ref[...] = zero
        rep_ref[...] = zero

    pt = ptile_ref[0]
    pc = [pcx_ref[0], pcy_ref[0], pcz_ref[0]]
    pr = [prx_ref[0], pry_ref[0], prz_ref[0]]
    gr = [grx_ref[0], gry_ref[0], grz_ref[0]]

    aa = jnp.sum(pt * pt, axis=1, keepdims=True)
    ciota = jax.lax.broadcasted_iota(jnp.int32, (_T, N), 1)

    def sel_matrix(full_ref, bb_ref):
        ab = jax.lax.dot_general(pt, full_ref[0], (((1,), (1,)), ((), ())),
                                 preferred_element_type=jnp.float32)
        return (aa - 2.0 * ab) + bb_ref[0]

    def exact_d2(rows):
        acc = None
        for c in range(3):
            t = (pc[c] - rows[c]) ** 2
            acc = t if acc is None else acc + t
        return acc

    def first_min_onehot(d):
        m = jnp.min(d, axis=1, keepdims=True)
        idx = jnp.min(jnp.where(d == m, ciota, N), axis=1, keepdims=True)
        return ciota == idx

    d_pg_sel = sel_matrix(gfull_ref, gbb_ref)
    oh = first_min_onehot(d_pg_sel)
    val = None
    for c in range(3):
        g_at = jnp.sum(jnp.where(oh, gr[c], 0.0), axis=1, keepdims=True)
        t = (pc[c] - g_at) ** 2
        val = t if val is None else val + t
    mnsum = jnp.sum(val, axis=0, keepdims=True)
    emd_ref[...] += mnsum * (100.0 / (3.0 * N * B)) / pcd_ref[0]

    d_pp_sel = sel_matrix(pfull_ref, pbb_ref)
    d_pp_exact = exact_d2(pr)
    rep_acc = jnp.zeros((1, 1), jnp.float32)
    for k in range(NN_SIZE):
        oh = first_min_onehot(d_pp_sel)
        if k > 0:
            val = jnp.sum(jnp.where(oh, d_pp_exact, 0.0),
                          axis=1, keepdims=True)
            d2c = jnp.maximum(val, EPS)
            dist = jnp.sqrt(d2c)
            w = jnp.exp(-d2c / (H * H))
            rep_acc = rep_acc + jnp.sum((RADIUS - dist) * w, axis=0,
                                        keepdims=True)
        if k < NN_SIZE - 1:
            d_pp_sel = jnp.where(oh, BIG, d_pp_sel)
    rep_ref[...] += rep_acc / (B * N * (NN_SIZE - 1))


def _oddeven_pairs(n):
    pairs = []
    p = 1
    while p < n:
        k = p
        while k >= 1:
            for j in range(k % p, n - k, 2 * k):
                for i in range(0, min(k, n - j - k)):
                    if (i + j) // (2 * p) == (i + j + k) // (2 * p):
                        pairs.append((i + j, i + j + k))
            k //= 2
        p *= 2
    return pairs


_PAIRS16 = _oddeven_pairs(16)


def _nsqrt(x):
    b = plsc.bitcast(x, jnp.int32)
    y = plsc.bitcast((b >> 1) + 0x1fbd1df5, jnp.float32)
    for _ in range(4):
        y = 0.5 * (y + x / y)
    return y


def _sorted_diag(buf):
    px = buf[0, :]
    py = buf[1, :]
    pz = buf[2, :]
    iot = jax.lax.iota(jnp.int32, 16)
    rows = []
    for i in range(N_TDA - 1):
        idx = jnp.full((16,), i, jnp.int32)
        pxi = px.at[idx].get(mode="promise_in_bounds")
        pyi = py.at[idx].get(mode="promise_in_bounds")
        pzi = pz.at[idx].get(mode="promise_in_bounds")
        dx = px - pxi
        dy = py - pyi
        dz = pz - pzi
        d2 = dx * dx + dy * dy + dz * dz
        dist = _nsqrt(jnp.maximum(d2, EPS))
        rows.append(jnp.where(iot > i, dist, BIG))
    rows.append(jnp.full((16,), BIG, jnp.float32))
    rows = [jnp.sort(r) for r in rows]
    for (a, b) in _PAIRS16:
        u, v = rows[a], rows[b]
        vr = jax.lax.rev(v, (0,))
        rows[a] = jnp.sort(jnp.minimum(u, vr))
        rows[b] = jnp.sort(jnp.maximum(u, vr))
    return rows


def _tda_body(p_hbm, g_hbm, out_hbm):
    c = jax.lax.axis_index("c")
    s = jax.lax.axis_index("s")

    def scoped(pbuf, gbuf, obuf):
        @pl.when((c == 0) & (s < 8))
        def _():
            pltpu.sync_copy(p_hbm.at[s], pbuf)
            pltpu.sync_copy(g_hbm.at[s], gbuf)
            dp = _sorted_diag(pbuf)
            dg = _sorted_diag(gbuf)
            acc = jnp.zeros((16,), jnp.float32)
            for k in range(16):
                d = dp[k] - dg[k]
                acc = acc + d * d
            sumsq = jnp.sum(acc)
            tda = _nsqrt(jnp.broadcast_to(sumsq + EPS, (16,)))
            obuf[...] = tda
            pltpu.sync_copy(obuf, out_hbm.at[s])

    pl.run_scoped(scoped,
                  pltpu.VMEM((3, 16), jnp.float32),
                  pltpu.VMEM((3, 16), jnp.float32),
                  pltpu.VMEM((16,), jnp.float32))


@jax.jit
def _run(pred, gt, pcd_radius):
    B, N, _ = pred.shape
    pcols = [pred[:, :, c:c + 1] for c in range(3)]
    prows = [jnp.swapaxes(p, 1, 2) for p in pcols]
    grows = [jnp.swapaxes(gt[:, :, c:c + 1], 1, 2) for c in range(3)]
    pbb = jnp.swapaxes(jnp.sum(pred * pred, axis=-1, keepdims=True), 1, 2)
    gbb = jnp.swapaxes(jnp.sum(gt * gt, axis=-1, keepdims=True), 1, 2)
    grid = (B, N // _T)
    tile_spec = pl.BlockSpec((1, _T, 3), lambda b, i: (b, i, 0))
    full_spec = pl.BlockSpec((1, N, 3), lambda b, i: (b, 0, 0))
    col_spec = pl.BlockSpec((1, _T, 1), lambda b, i: (b, i, 0))
    row_spec = pl.BlockSpec((1, 1, N), lambda b, i: (b, 0, 0))
    scalar_spec = pl.BlockSpec((1, 1), lambda b, i: (0, 0))
    emd, rep = pl.pallas_call(
        _loss_kernel,
        grid=grid,
        in_specs=[tile_spec, full_spec, full_spec] +
                 [col_spec] * 3 + [row_spec] * 6 + [row_spec] * 2 +
                 [pl.BlockSpec((1, 1, 1), lambda b, i: (b, 0, 0))],
        out_specs=[scalar_spec] * 2,
        out_shape=[jax.ShapeDtypeStruct((1, 1), jnp.float32)] * 2,
    )(pred, pred, gt, *pcols, *prows, *grows, pbb, gbb,
      pcd_radius.reshape(B, 1, 1))

    p16t = jnp.swapaxes(pred[:, :N_TDA, :], 1, 2)
    g16t = jnp.swapaxes(gt[:, :N_TDA, :], 1, 2)
    tda_rows = pl.kernel(
        _tda_body,
        out_type=jax.ShapeDtypeStruct((8, 16), jnp.float32),
        mesh=plsc.VectorSubcoreMesh(core_axis_name="c",
                                    subcore_axis_name="s"),
        compiler_params=pltpu.CompilerParams(needs_layout_passes=False),
    )(p16t, g16t)
    tda = jnp.mean(tda_rows[:, 0])

    return (emd[0, 0], rep[0, 0] * ALPHA, tda * BETA)


def kernel(pred, gt, pcd_radius):
    return _run(pred, gt, pcd_radius)


# speedup vs baseline: 1.0839x; 1.0449x over previous
import functools

import jax
import jax.numpy as jnp
import numpy as np
from jax.experimental import pallas as pl
import jax.experimental.pallas.tpu as pltpu
from jax.experimental.pallas import tpu_sc as plsc

ALPHA = 1.0
BETA = 1.0
N_TDA = 16
Q = 2
NN_SIZE = 5
RADIUS = 0.07
H = 0.03
EPS = 1e-12
BIG = 1e9

_T = 512


def _loss_kernel(ptile_ref, pfull_ref, gfull_ref,
                 pcx_ref, pcy_ref, pcz_ref,
                 prx_ref, pry_ref, prz_ref,
                 grx_ref, gry_ref, grz_ref,
                 pbb_ref, gbb_ref, pcd_ref,
                 emd_ref, rep_ref):
    b = pl.program_id(0)
    i = pl.program_id(1)
    B = pl.num_programs(0)
    NT = pl.num_programs(1)
    N = NT * _T

    @pl.when((b == 0) & (i == 0))
    def _init():
        zero = jnp.zeros((1, 1), jnp.float32)
        emd_ref[...] = zero
        rep_ref[...] = zero

    pt = ptile_ref[0]
    pc = [pcx_ref[0], pcy_ref[0], pcz_ref[0]]
    pr = [prx_ref[0], pry_ref[0], prz_ref[0]]
    gr = [grx_ref[0], gry_ref[0], grz_ref[0]]

    aa = jnp.sum(pt * pt, axis=1, keepdims=True)
    ciota = jax.lax.broadcasted_iota(jnp.int32, (_T, N), 1)

    def sel_matrix(full_ref, bb_ref):
        ab = jax.lax.dot_general(pt, full_ref[0], (((1,), (1,)), ((), ())),
                                 preferred_element_type=jnp.float32)
        return (aa - 2.0 * ab) + bb_ref[0]

    def exact_d2(rows):
        acc = None
        for c in range(3):
            t = (pc[c] - rows[c]) ** 2
            acc = t if acc is None else acc + t
        return acc

    def first_min_onehot(d):
        m = jnp.min(d, axis=1, keepdims=True)
        idx = jnp.min(jnp.where(d == m, ciota, N), axis=1, keepdims=True)
        return ciota == idx

    d_pg_sel = sel_matrix(gfull_ref, gbb_ref)
    oh = first_min_onehot(d_pg_sel)
    val = None
    for c in range(3):
        g_at = jnp.sum(jnp.where(oh, gr[c], 0.0), axis=1, keepdims=True)
        t = (pc[c] - g_at) ** 2
        val = t if val is None else val + t
    mnsum = jnp.sum(val, axis=0, keepdims=True)
    emd_ref[...] += mnsum * (100.0 / (3.0 * N * B)) / pcd_ref[0]

    d_pp_sel = sel_matrix(pfull_ref, pbb_ref)
    d_pp_exact = exact_d2(pr)
    rep_acc = jnp.zeros((1, 1), jnp.float32)
    for k in range(NN_SIZE):
        oh = first_min_onehot(d_pp_sel)
        if k > 0:
            val = jnp.sum(jnp.where(oh, d_pp_exact, 0.0),
                          axis=1, keepdims=True)
            d2c = jnp.maximum(val, EPS)
            dist = jnp.sqrt(d2c)
            w = jnp.exp(-d2c / (H * H))
            rep_acc = rep_acc + jnp.sum((RADIUS - dist) * w, axis=0,
                                        keepdims=True)
        if k < NN_SIZE - 1:
            d_pp_sel = jnp.where(oh, BIG, d_pp_sel)
    rep_ref[...] += rep_acc / (B * N * (NN_SIZE - 1))


def _oddeven_pairs(n):
    pairs = []
    p = 1
    while p < n:
        k = p
        while k >= 1:
            for j in range(k % p, n - k, 2 * k):
                for i in range(0, min(k, n - j - k)):
                    if (i + j) // (2 * p) == (i + j + k) // (2 * p):
                        pairs.append((i + j, i + j + k))
            k //= 2
        p *= 2
    return pairs


_PAIRS16 = _oddeven_pairs(16)


def _nsqrt(x):
    b = plsc.bitcast(x, jnp.int32)
    y = plsc.bitcast((b >> 1) + 0x1fbd1df5, jnp.float32)
    for _ in range(4):
        y = 0.5 * (y + x / y)
    return y


def _sorted_diag(buf):
    px = buf[0, :]
    py = buf[1, :]
    pz = buf[2, :]
    iot = jax.lax.iota(jnp.int32, 16)
    rows = []
    for i in range(N_TDA - 1):
        pxi = jnp.sum(jnp.where(iot == i, px, 0.0))
        pyi = jnp.sum(jnp.where(iot == i, py, 0.0))
        pzi = jnp.sum(jnp.where(iot == i, pz, 0.0))
        dx = px - pxi
        dy = py - pyi
        dz = pz - pzi
        d2 = dx * dx + dy * dy + dz * dz
        dist = _nsqrt(jnp.maximum(d2, EPS))
        rows.append(jnp.where(iot > i, dist, BIG))
    rows.append(jnp.full((16,), BIG, jnp.float32))
    rows = [jnp.sort(r) for r in rows]
    for (a, b) in _PAIRS16:
        u, v = rows[a], rows[b]
        vr = -jnp.sort(-v)
        rows[a] = jnp.sort(jnp.minimum(u, vr))
        rows[b] = jnp.sort(jnp.maximum(u, vr))
    return rows


def _tda_body(p_hbm, g_hbm, out_hbm):
    c = jax.lax.axis_index("c")
    s = jax.lax.axis_index("s")

    def scoped(pbuf, gbuf, obuf):
        @pl.when((c == 0) & (s < 8))
        def _():
            pltpu.sync_copy(p_hbm.at[s], pbuf)
            pltpu.sync_copy(g_hbm.at[s], gbuf)
            dp = _sorted_diag(pbuf)
            dg = _sorted_diag(gbuf)
            acc = jnp.zeros((16,), jnp.float32)
            for k in range(16):
                d = dp[k] - dg[k]
                acc = acc + d * d
            sumsq = jnp.sum(acc)
            tda = _nsqrt(jnp.broadcast_to(sumsq + EPS, (16,)))
            obuf[...] = tda
            pltpu.sync_copy(obuf, out_hbm.at[s])

    pl.run_scoped(scoped,
                  pltpu.VMEM((3, 16), jnp.float32),
                  pltpu.VMEM((3, 16), jnp.float32),
                  pltpu.VMEM((16,), jnp.float32))


@jax.jit
def _run(pred, gt, pcd_radius):
    B, N, _ = pred.shape
    pcols = [pred[:, :, c:c + 1] for c in range(3)]
    prows = [jnp.swapaxes(p, 1, 2) for p in pcols]
    grows = [jnp.swapaxes(gt[:, :, c:c + 1], 1, 2) for c in range(3)]
    pbb = jnp.swapaxes(jnp.sum(pred * pred, axis=-1, keepdims=True), 1, 2)
    gbb = jnp.swapaxes(jnp.sum(gt * gt, axis=-1, keepdims=True), 1, 2)
    grid = (B, N // _T)
    tile_spec = pl.BlockSpec((1, _T, 3), lambda b, i: (b, i, 0))
    full_spec = pl.BlockSpec((1, N, 3), lambda b, i: (b, 0, 0))
    col_spec = pl.BlockSpec((1, _T, 1), lambda b, i: (b, i, 0))
    row_spec = pl.BlockSpec((1, 1, N), lambda b, i: (b, 0, 0))
    scalar_spec = pl.BlockSpec((1, 1), lambda b, i: (0, 0))
    emd, rep = pl.pallas_call(
        _loss_kernel,
        grid=grid,
        in_specs=[tile_spec, full_spec, full_spec] +
                 [col_spec] * 3 + [row_spec] * 6 + [row_spec] * 2 +
                 [pl.BlockSpec((1, 1, 1), lambda b, i: (b, 0, 0))],
        out_specs=[scalar_spec] * 2,
        out_shape=[jax.ShapeDtypeStruct((1, 1), jnp.float32)] * 2,
    )(pred, pred, gt, *pcols, *prows, *grows, pbb, gbb,
      pcd_radius.reshape(B, 1, 1))

    p16t = jnp.swapaxes(pred[:, :N_TDA, :], 1, 2)
    g16t = jnp.swapaxes(gt[:, :N_TDA, :], 1, 2)
    tda_rows = pl.kernel(
        _tda_body,
        out_type=jax.ShapeDtypeStruct((8, 16), jnp.float32),
        mesh=plsc.VectorSubcoreMesh(core_axis_name="c",
                                    subcore_axis_name="s"),
        compiler_params=pltpu.CompilerParams(needs_layout_passes=False),
    )(p16t, g16t)
    tda = jnp.mean(tda_rows[:, 0])

    return (emd[0, 0], rep[0, 0] * ALPHA, tda * BETA)


def kernel(pred, gt, pcd_radius):
    return _run(pred, gt, pcd_radius)
